# baseline (speedup 1.0000x reference)
import functools

import numpy as np
import jax
import jax.numpy as jnp
from jax import lax
from jax.experimental import pallas as pl
from jax.experimental.pallas import tpu as pltpu
from jax.experimental.pallas import tpu_sc as plsc

VOCAB = 100000
D_MODEL = 1024
MAX_LEN = 2048
BATCH = 4
SEQ = 2048

NC, NS = 2, 16
NW = NC * NS
LANES = 16
TOTAL = BATCH * SEQ
PER_W = TOTAL // NW
CHUNK = 32
N_CHUNKS = PER_W // CHUNK
SCALE = float(D_MODEL) ** 0.5


def _make_pe(max_len, d_model):
    pe = np.zeros((max_len, d_model), dtype=np.float32)
    position = np.arange(0, max_len, dtype=np.float32)[:, None]
    div_term = np.exp(
        np.arange(0, d_model, 2, dtype=np.float32) * -(np.log(10000.0) / d_model))
    pe[:, 0::2] = np.sin(position * div_term)
    pe[:, 1::2] = np.cos(position * div_term)
    return pe


_PE = _make_pe(MAX_LEN, D_MODEL)


def _sc_embed(x_flat, table, pe):
    mesh = plsc.VectorSubcoreMesh(core_axis_name="c", subcore_axis_name="s")

    @functools.partial(
        pl.kernel,
        out_type=jax.ShapeDtypeStruct((TOTAL, D_MODEL), jnp.float32),
        mesh=mesh,
        scratch_types=[
            pltpu.VMEM((PER_W,), jnp.int32),
            pltpu.VMEM((CHUNK, D_MODEL), jnp.float32),
            pltpu.VMEM((CHUNK, D_MODEL), jnp.float32),
            pltpu.SemaphoreType.DMA,
        ],
    )
    def k(x_hbm, table_hbm, pe_hbm, out_hbm, idx_v, rows_v, pe_v, sem):
        wid = lax.axis_index("s") * NC + lax.axis_index("c")
        base = wid * PER_W
        s0 = base % SEQ

        pltpu.sync_copy(x_hbm.at[pl.ds(base, PER_W)], idx_v)

        def chunk_body(c, _):
            row0 = c * CHUNK
            gather = pltpu.async_copy(
                table_hbm.at[idx_v.at[pl.ds(row0, CHUNK)]], rows_v, sem)
            pltpu.sync_copy(pe_hbm.at[pl.ds(s0 + row0, CHUNK)], pe_v)
            gather.wait()

            def row_body(r, _):
                for g in range(D_MODEL // LANES):
                    sl = pl.ds(g * LANES, LANES)
                    rows_v[r, sl] = rows_v[r, sl] * SCALE + pe_v[r, sl]
                return 0

            lax.fori_loop(0, CHUNK, row_body, 0)
            pltpu.sync_copy(rows_v, out_hbm.at[pl.ds(base + row0, CHUNK)])
            return 0

        lax.fori_loop(0, N_CHUNKS, chunk_body, 0)

    return k(x_flat, table, pe)


def kernel(x, table):
    x_flat = jnp.reshape(x, (TOTAL,)).astype(jnp.int32)
    out = _sc_embed(x_flat, table, _PE)
    return jnp.reshape(out, (BATCH, SEQ, D_MODEL))
